# Initial kernel scaffold; baseline (speedup 1.0000x reference)
import jax
import jax.numpy as jnp
from jax.experimental import pallas as pl


def kernel(x, sx, edge_index, edge_attr, batch, tW1, tb1, tW2, tb2, s1W1, s1b1, s1W2, s1b2, s2W1, s2b1, s2W2, s2b2, pW1, pb1, pW2, pb2):
    raise NotImplementedError("write your pallas kernel here")


# speedup vs baseline: 2.2614x; 2.2614x over previous
import functools

import jax
import jax.numpy as jnp
from jax import lax
from jax.experimental import pallas as pl
from jax.experimental.pallas import tpu as pltpu
from jax.experimental.pallas import tpu_sc as plsc

_NC = 2
_NS = 16
_NW = _NC * _NS


def _mesh():
    return plsc.VectorSubcoreMesh(core_axis_name="c", subcore_axis_name="s")


def _sc_gather(table, idx, ch):
    n, c = table.shape
    e = idx.shape[0]
    per_w = e // _NW
    assert per_w * _NW == e and per_w % ch == 0
    ng = per_w // ch

    @functools.partial(
        pl.kernel,
        out_type=jax.ShapeDtypeStruct((e, c), jnp.float32),
        mesh=_mesh(),
        compiler_params=pltpu.CompilerParams(use_tc_tiling_on_sc=False),
        scratch_types=[
            pltpu.VMEM((ch,), jnp.int32),
            pltpu.VMEM((ch,), jnp.int32),
            pltpu.VMEM((ch, c), jnp.float32),
            pltpu.VMEM((ch, c), jnp.float32),
            pltpu.SemaphoreType.DMA, pltpu.SemaphoreType.DMA,
            pltpu.SemaphoreType.DMA, pltpu.SemaphoreType.DMA,
            pltpu.SemaphoreType.DMA, pltpu.SemaphoreType.DMA,
        ],
    )
    def k(tab_hbm, idx_hbm, out_hbm, ib0, ib1, rb0, rb1,
          is0, is1, gs0, gs1, ss0, ss1):
        cc = lax.axis_index("c")
        ss = lax.axis_index("s")
        base = (ss * _NC + cc) * per_w
        ib = (ib0, ib1)
        rb = (rb0, rb1)
        isem = (is0, is1)
        gsem = (gs0, gs1)
        ssem = (ss0, ss1)

        def load_idx(g, b):
            return pltpu.async_copy(
                idx_hbm.at[pl.ds(base + g * ch, ch)], ib[b], isem[b])

        def start_gather(b):
            return pltpu.async_copy(tab_hbm.at[ib[b]], rb[b], gsem[b])

        def start_store(g, b):
            return pltpu.async_copy(
                rb[b], out_hbm.at[pl.ds(base + g * ch, ch)], ssem[b])

        pend_i = {0: load_idx(0, 0)}
        if ng > 1:
            pend_i[1] = load_idx(1, 1)
        pend_i[0].wait()
        pend_g = {0: start_gather(0)}
        pend_s = {}
        for g in range(ng):
            b = g & 1
            pend_g[g].wait()
            if g + 2 < ng:
                pend_i[g + 2] = load_idx(g + 2, b)
            if g + 1 < ng:
                pend_i[g + 1].wait()
                if g >= 1:
                    pend_s[g - 1].wait()
                pend_g[g + 1] = start_gather(1 - b)
            pend_s[g] = start_store(g, b)
        pend_s[ng - 1].wait()
        if ng > 1:
            pend_s[ng - 2].wait()

    return k(table, idx)


def _sc_degree(dst, ones_hbm, zeros_hbm, n):
    e = dst.shape[0]
    ch = 1000
    per_t = e // _NW
    assert per_t % ch == 0
    ng = per_t // ch
    zch = zeros_hbm.shape[0]
    nz = n // zch
    assert nz * zch == n

    @functools.partial(
        pl.kernel,
        out_type=[jax.ShapeDtypeStruct((n,), jnp.float32),
                  jax.ShapeDtypeStruct((n,), jnp.float32)],
        mesh=_mesh(),
        scratch_types=[
            pltpu.VMEM_SHARED((n,), jnp.float32),
            pltpu.VMEM((ch,), jnp.float32),
            pltpu.VMEM((zch,), jnp.float32),
            pltpu.VMEM((ch,), jnp.int32),
            pltpu.VMEM((ch,), jnp.int32),
            pltpu.SemaphoreType.DMA, pltpu.SemaphoreType.DMA,
        ],
    )
    def k(dst_hbm, ones_h, zero_h, out0_hbm, out1_hbm, acc, ones_v, stage,
          ib0, ib1, is0, is1):
        cc = lax.axis_index("c")
        tt = lax.axis_index("s")
        pltpu.sync_copy(ones_h, ones_v)
        pltpu.sync_copy(zero_h, stage)
        for j in range((nz + _NS - 1) // _NS):
            kk = tt + _NS * j

            @pl.when(kk < nz)
            def _():
                pltpu.sync_copy(stage, acc.at[pl.ds(kk * zch, zch)])
        plsc.subcore_barrier()

        base = cc * (e // _NC) + tt * per_t
        ib = (ib0, ib1)
        isem = (is0, is1)

        def load_idx(g, b):
            return pltpu.async_copy(
                dst_hbm.at[pl.ds(base + g * ch, ch)], ib[b], isem[b])

        pend = {0: load_idx(0, 0)}
        if ng > 1:
            pend[1] = load_idx(1, 1)
        for g in range(ng):
            b = g & 1
            pend[g].wait()
            pltpu.sync_copy(ones_v, acc.at[ib[b]], add=True)
            if g + 2 < ng:
                pend[g + 2] = load_idx(g + 2, b)
        plsc.subcore_barrier()
        for j in range((nz + _NS - 1) // _NS):
            kk = tt + _NS * j

            @pl.when(jnp.logical_and(kk < nz, cc == 0))
            def _():
                pltpu.sync_copy(acc.at[pl.ds(kk * zch, zch)], stage)
                pltpu.sync_copy(stage, out0_hbm.at[pl.ds(kk * zch, zch)])

            @pl.when(jnp.logical_and(kk < nz, cc == 1))
            def _():
                pltpu.sync_copy(acc.at[pl.ds(kk * zch, zch)], stage)
                pltpu.sync_copy(stage, out1_hbm.at[pl.ds(kk * zch, zch)])

    return k(dst, ones_hbm, zeros_hbm)


def _sc_scatter(m, dst, zrows_hbm, n):
    e = dst.shape[0]
    hc = m.shape[2]
    ch = 400
    per_t = e // _NS
    assert per_t % ch == 0
    ng = per_t // ch
    zch = zrows_hbm.shape[0]
    nz = n // zch
    assert nz * zch == n

    @functools.partial(
        pl.kernel,
        out_type=jax.ShapeDtypeStruct((_NC, n, hc), jnp.float32),
        mesh=_mesh(),
        compiler_params=pltpu.CompilerParams(use_tc_tiling_on_sc=False),
        scratch_types=[
            pltpu.VMEM_SHARED((n, hc), jnp.float32),
            pltpu.VMEM((ch,), jnp.int32),
            pltpu.VMEM((ch,), jnp.int32),
            pltpu.VMEM((ch, hc), jnp.float32),
            pltpu.VMEM((ch, hc), jnp.float32),
            pltpu.SemaphoreType.DMA, pltpu.SemaphoreType.DMA,
            pltpu.SemaphoreType.DMA, pltpu.SemaphoreType.DMA,
        ],
    )
    def k(m_hbm, dst_hbm, zrows_h, out_hbm, acc, ib0, ib1, ub0, ub1,
          is0, is1, us0, us1):
        cc = lax.axis_index("c")
        tt = lax.axis_index("s")
        pltpu.sync_copy(zrows_h, ub0)
        for j in range((nz + _NS - 1) // _NS):
            kk = tt + _NS * j

            @pl.when(kk < nz)
            def _():
                pltpu.sync_copy(ub0, acc.at[pl.ds(kk * zch, zch)])
        plsc.subcore_barrier()

        base = tt * per_t
        ib = (ib0, ib1)
        ub = (ub0, ub1)
        isem = (is0, is1)
        usem = (us0, us1)

        def load(g, b):
            di = pltpu.async_copy(
                dst_hbm.at[pl.ds(base + g * ch, ch)], ib[b], isem[b])
            du = pltpu.async_copy(
                m_hbm.at[cc, pl.ds(base + g * ch, ch)], ub[b], usem[b])
            return di, du

        pend = {0: load(0, 0)}
        if ng > 1:
            pend[1] = load(1, 1)
        for g in range(ng):
            b = g & 1
            pend[g][0].wait()
            pend[g][1].wait()
            pltpu.sync_copy(ub[b], acc.at[ib[b]], add=True)
            if g + 2 < ng:
                pend[g + 2] = load(g + 2, b)
        plsc.subcore_barrier()
        for j in range((nz + _NS - 1) // _NS):
            kk = tt + _NS * j

            @pl.when(kk < nz)
            def _():
                pltpu.sync_copy(acc.at[pl.ds(kk * zch, zch)], ub0)
                pltpu.sync_copy(ub0, out_hbm.at[cc, pl.ds(kk * zch, zch)])

    return k(m, dst, zrows_hbm)


def _dot(a, b):
    return jax.lax.dot_general(a, b, (((1,), (0,)), ((), ())),
                               preferred_element_type=jnp.float32)


def _full(shape):
    return pl.BlockSpec(shape, lambda i: tuple(0 for _ in shape))


def _prep1(x, tW1, tb1, tW2, tb2, w1a, b1, bn):
    n = x.shape[0]

    def body(x_ref, tw1, tb1r, tw2, tb2r, wa, b1r, x0_ref, g1_ref):
        x0 = jnp.maximum(_dot(x_ref[...], tw1[...]) + tb1r[...], 0.0)
        x0 = jnp.maximum(_dot(x0, tw2[...]) + tb2r[...], 0.0)
        x0_ref[...] = x0
        g1_ref[...] = _dot(x0, wa[...]) + b1r[...]

    return pl.pallas_call(
        body,
        grid=(n // bn,),
        in_specs=[
            pl.BlockSpec((bn, x.shape[1]), lambda i: (i, 0)),
            _full(tW1.shape), _full(tb1.shape), _full(tW2.shape),
            _full(tb2.shape), _full(w1a.shape), _full(b1.shape),
        ],
        out_specs=[
            pl.BlockSpec((bn, 32), lambda i: (i, 0)),
            pl.BlockSpec((bn, 32), lambda i: (i, 0)),
        ],
        out_shape=[
            jax.ShapeDtypeStruct((n, 32), jnp.float32),
            jax.ShapeDtypeStruct((n, 32), jnp.float32),
        ],
    )(x, tW1, tb1, tW2, tb2, w1a, b1)


def _edge_mlp(garr, ea, w1e, w2, b2, be):
    e, c = garr.shape

    def body(g_ref, ea_ref, we, w2r, b2r, out_ref):
        h = jnp.maximum(g_ref[...] + _dot(ea_ref[...], we[...]), 0.0)
        m = jnp.maximum(_dot(h, w2r[...]) + b2r[...], 0.0)
        out_ref[0] = m[:, :32]
        out_ref[1] = m[:, 32:]

    return pl.pallas_call(
        body,
        grid=(e // be,),
        in_specs=[
            pl.BlockSpec((be, c), lambda i: (i, 0)),
            pl.BlockSpec((be, 16), lambda i: (i, 0)),
            _full(w1e.shape), _full(w2.shape), _full(b2.shape),
        ],
        out_specs=pl.BlockSpec((2, be, 32), lambda i: (0, i, 0)),
        out_shape=jax.ShapeDtypeStruct((2, e, 32), jnp.float32),
    )(garr, ea, w1e, w2, b2)


def _mid(agg, deg0, deg1, x0, wlo, whi, wx, b1, bn):
    n = x0.shape[0]

    def body(a_ref, d0, d1, x0_ref, wl, wh, wxr, b1r, out_ref):
        inv = 1.0 / jnp.maximum(d0[...] + d1[...], 1.0)
        lo = a_ref[0] * inv
        hi = a_ref[1] * inv
        out_ref[...] = (_dot(lo, wl[...]) + _dot(hi, wh[...])
                        + _dot(x0_ref[...], wxr[...]) + b1r[...])

    return pl.pallas_call(
        body,
        grid=(n // bn,),
        in_specs=[
            pl.BlockSpec((2, bn, 32), lambda i: (0, i, 0)),
            pl.BlockSpec((bn, 1), lambda i: (i, 0)),
            pl.BlockSpec((bn, 1), lambda i: (i, 0)),
            pl.BlockSpec((bn, 32), lambda i: (i, 0)),
            _full(wlo.shape), _full(whi.shape), _full(wx.shape),
            _full(b1.shape),
        ],
        out_specs=pl.BlockSpec((bn, 64), lambda i: (i, 0)),
        out_shape=jax.ShapeDtypeStruct((n, 64), jnp.float32),
    )(agg, deg0, deg1, x0, wlo, whi, wx, b1)


def _head(agg, deg0, deg1, x0, plo, phi, px, pb1, pw2, pb2, bn):
    n = x0.shape[0]

    def body(a_ref, d0, d1, x0_ref, wl, wh, wxr, b1r, w2r, b2r, out_ref):
        inv = 1.0 / jnp.maximum(d0[...] + d1[...], 1.0)
        lo = a_ref[0] * inv
        hi = a_ref[1] * inv
        h = jnp.maximum(_dot(lo, wl[...]) + _dot(hi, wh[...])
                        + _dot(x0_ref[...], wxr[...]) + b1r[...], 0.0)
        out_ref[...] = _dot(h, w2r[...]) + b2r[...]

    return pl.pallas_call(
        body,
        grid=(n // bn,),
        in_specs=[
            pl.BlockSpec((2, bn, 32), lambda i: (0, i, 0)),
            pl.BlockSpec((bn, 1), lambda i: (i, 0)),
            pl.BlockSpec((bn, 1), lambda i: (i, 0)),
            pl.BlockSpec((bn, 32), lambda i: (i, 0)),
            _full(plo.shape), _full(phi.shape), _full(px.shape),
            _full(pb1.shape), _full(pw2.shape), _full(pb2.shape),
        ],
        out_specs=pl.BlockSpec((bn, 1), lambda i: (i, 0)),
        out_shape=jax.ShapeDtypeStruct((n, 1), jnp.float32),
    )(agg, deg0, deg1, x0, plo, phi, px, pb1, pw2, pb2)


def kernel(x, sx, edge_index, edge_attr, batch, tW1, tb1, tW2, tb2,
           s1W1, s1b1, s1W2, s1b2, s2W1, s2b1, s2W2, s2b2,
           pW1, pb1, pW2, pb2):
    n = x.shape[0]
    src = edge_index[0]
    dst = edge_index[1]

    bn = 2000
    be = 4000

    ones1 = jnp.ones((1000,), jnp.float32)
    z1 = jnp.zeros((2000,), jnp.float32)
    z32 = jnp.zeros((400, 32), jnp.float32)

    r = lambda v: v.reshape(1, -1)

    degp0, degp1 = _sc_degree(dst, ones1, z1, n)
    deg0 = degp0.reshape(n, 1)
    deg1 = degp1.reshape(n, 1)

    x0, g1 = _prep1(x, tW1, r(tb1), tW2, r(tb2), s1W1[:32], r(s1b1), bn)

    garr1 = _sc_gather(g1, src, 1000)
    m1 = _edge_mlp(garr1, edge_attr, s1W1[32:48], s1W2, r(s1b2), be)
    agg1 = _sc_scatter(m1, dst, z32, n)

    g2 = _mid(agg1, deg0, deg1, x0, s2W1[:32], s2W1[32:64], s2W1[64:96],
              r(s2b1), bn)

    garr2 = _sc_gather(g2, src, 1000)
    m2 = _edge_mlp(garr2, edge_attr, s2W1[96:112], s2W2, r(s2b2), be)
    agg2 = _sc_scatter(m2, dst, z32, n)

    return _head(agg2, deg0, deg1, x0, pW1[:32], pW1[32:64], pW1[64:96],
                 r(pb1), pW2, r(pb2), bn)


# speedup vs baseline: 2.7714x; 1.2256x over previous
import functools

import jax
import jax.numpy as jnp
from jax import lax
from jax.experimental import pallas as pl
from jax.experimental.pallas import tpu as pltpu
from jax.experimental.pallas import tpu_sc as plsc

_NC = 2
_NS = 16
_NW = _NC * _NS


def _mesh():
    return plsc.VectorSubcoreMesh(core_axis_name="c", subcore_axis_name="s")


def _sc_gather(table, idx, ch):
    n, c = table.shape
    e = idx.shape[0]
    per_w = e // _NW
    assert per_w * _NW == e and per_w % ch == 0
    ng = per_w // ch

    @functools.partial(
        pl.kernel,
        out_type=jax.ShapeDtypeStruct((e, c), jnp.float32),
        mesh=_mesh(),
        compiler_params=pltpu.CompilerParams(use_tc_tiling_on_sc=False),
        scratch_types=[
            pltpu.VMEM((ch,), jnp.int32),
            pltpu.VMEM((ch,), jnp.int32),
            pltpu.VMEM((ch, c), jnp.float32),
            pltpu.VMEM((ch, c), jnp.float32),
            pltpu.SemaphoreType.DMA, pltpu.SemaphoreType.DMA,
            pltpu.SemaphoreType.DMA, pltpu.SemaphoreType.DMA,
            pltpu.SemaphoreType.DMA, pltpu.SemaphoreType.DMA,
        ],
    )
    def k(tab_hbm, idx_hbm, out_hbm, ib0, ib1, rb0, rb1,
          is0, is1, gs0, gs1, ss0, ss1):
        cc = lax.axis_index("c")
        ss = lax.axis_index("s")
        base = (ss * _NC + cc) * per_w
        ib = (ib0, ib1)
        rb = (rb0, rb1)
        isem = (is0, is1)
        gsem = (gs0, gs1)
        ssem = (ss0, ss1)

        def load_idx(g, b):
            return pltpu.async_copy(
                idx_hbm.at[pl.ds(base + g * ch, ch)], ib[b], isem[b])

        def start_gather(b):
            return pltpu.async_copy(tab_hbm.at[ib[b]], rb[b], gsem[b])

        def start_store(g, b):
            return pltpu.async_copy(
                rb[b], out_hbm.at[pl.ds(base + g * ch, ch)], ssem[b])

        pend_i = {0: load_idx(0, 0)}
        if ng > 1:
            pend_i[1] = load_idx(1, 1)
        pend_i[0].wait()
        pend_g = {0: start_gather(0)}
        pend_s = {}
        for g in range(ng):
            b = g & 1
            pend_g[g].wait()
            if g + 2 < ng:
                pend_i[g + 2] = load_idx(g + 2, b)
            if g + 1 < ng:
                pend_i[g + 1].wait()
                if g >= 1:
                    pend_s[g - 1].wait()
                pend_g[g + 1] = start_gather(1 - b)
            pend_s[g] = start_store(g, b)
        pend_s[ng - 1].wait()
        if ng > 1:
            pend_s[ng - 2].wait()

    return k(table, idx)


def _sc_degree(dst, ones_hbm, zeros_hbm, n):
    e = dst.shape[0]
    ch = 1000
    per_t = e // _NW
    assert per_t % ch == 0
    ng = per_t // ch
    zch = zeros_hbm.shape[0]
    nz = n // zch
    assert nz * zch == n

    @functools.partial(
        pl.kernel,
        out_type=[jax.ShapeDtypeStruct((n,), jnp.float32),
                  jax.ShapeDtypeStruct((n,), jnp.float32)],
        mesh=_mesh(),
        scratch_types=[
            pltpu.VMEM_SHARED((n,), jnp.float32),
            pltpu.VMEM((ch,), jnp.float32),
            pltpu.VMEM((zch,), jnp.float32),
            pltpu.VMEM((ch,), jnp.int32),
            pltpu.VMEM((ch,), jnp.int32),
            pltpu.SemaphoreType.DMA, pltpu.SemaphoreType.DMA,
        ],
    )
    def k(dst_hbm, ones_h, zero_h, out0_hbm, out1_hbm, acc, ones_v, stage,
          ib0, ib1, is0, is1):
        cc = lax.axis_index("c")
        tt = lax.axis_index("s")
        pltpu.sync_copy(ones_h, ones_v)
        pltpu.sync_copy(zero_h, stage)
        for j in range((nz + _NS - 1) // _NS):
            kk = tt + _NS * j

            @pl.when(kk < nz)
            def _():
                pltpu.sync_copy(stage, acc.at[pl.ds(kk * zch, zch)])
        plsc.subcore_barrier()

        base = cc * (e // _NC) + tt * per_t
        ib = (ib0, ib1)
        isem = (is0, is1)

        def load_idx(g, b):
            return pltpu.async_copy(
                dst_hbm.at[pl.ds(base + g * ch, ch)], ib[b], isem[b])

        pend = {0: load_idx(0, 0)}
        if ng > 1:
            pend[1] = load_idx(1, 1)
        for g in range(ng):
            b = g & 1
            pend[g].wait()
            pltpu.sync_copy(ones_v, acc.at[ib[b]], add=True)
            if g + 2 < ng:
                pend[g + 2] = load_idx(g + 2, b)
        plsc.subcore_barrier()
        for j in range((nz + _NS - 1) // _NS):
            kk = tt + _NS * j

            @pl.when(jnp.logical_and(kk < nz, cc == 0))
            def _():
                pltpu.sync_copy(acc.at[pl.ds(kk * zch, zch)], stage)
                pltpu.sync_copy(stage, out0_hbm.at[pl.ds(kk * zch, zch)])

            @pl.when(jnp.logical_and(kk < nz, cc == 1))
            def _():
                pltpu.sync_copy(acc.at[pl.ds(kk * zch, zch)], stage)
                pltpu.sync_copy(stage, out1_hbm.at[pl.ds(kk * zch, zch)])

    return k(dst, ones_hbm, zeros_hbm)


def _sc_scatter(m, dst, zrows_hbm, n):
    e = dst.shape[0]
    hc = m.shape[2]
    ch = 400
    per_t = e // _NS
    assert per_t % ch == 0
    ng = per_t // ch
    zch = zrows_hbm.shape[0]
    nz = n // zch
    assert nz * zch == n

    @functools.partial(
        pl.kernel,
        out_type=jax.ShapeDtypeStruct((_NC, n, hc), jnp.float32),
        mesh=_mesh(),
        compiler_params=pltpu.CompilerParams(use_tc_tiling_on_sc=False),
        scratch_types=[
            pltpu.VMEM_SHARED((n, hc), jnp.float32),
            pltpu.VMEM((ch,), jnp.int32),
            pltpu.VMEM((ch,), jnp.int32),
            pltpu.VMEM((ch, hc), jnp.float32),
            pltpu.VMEM((ch, hc), jnp.float32),
            pltpu.SemaphoreType.DMA, pltpu.SemaphoreType.DMA,
            pltpu.SemaphoreType.DMA, pltpu.SemaphoreType.DMA,
        ],
    )
    def k(m_hbm, dst_hbm, zrows_h, out_hbm, acc, ib0, ib1, ub0, ub1,
          is0, is1, us0, us1):
        cc = lax.axis_index("c")
        tt = lax.axis_index("s")
        pltpu.sync_copy(zrows_h, ub0)
        for j in range((nz + _NS - 1) // _NS):
            kk = tt + _NS * j

            @pl.when(kk < nz)
            def _():
                pltpu.sync_copy(ub0, acc.at[pl.ds(kk * zch, zch)])
        plsc.subcore_barrier()

        base = tt * per_t
        ib = (ib0, ib1)
        ub = (ub0, ub1)
        isem = (is0, is1)
        usem = (us0, us1)

        def load(g, b):
            di = pltpu.async_copy(
                dst_hbm.at[pl.ds(base + g * ch, ch)], ib[b], isem[b])
            du = pltpu.async_copy(
                m_hbm.at[cc, pl.ds(base + g * ch, ch)], ub[b], usem[b])
            return di, du

        pend = {0: load(0, 0)}
        if ng > 1:
            pend[1] = load(1, 1)
        for g in range(ng):
            b = g & 1
            pend[g][0].wait()
            pend[g][1].wait()
            pltpu.sync_copy(ub[b], acc.at[ib[b]], add=True)
            if g + 2 < ng:
                pend[g + 2] = load(g + 2, b)
        plsc.subcore_barrier()
        for j in range((nz + _NS - 1) // _NS):
            kk = tt + _NS * j

            @pl.when(kk < nz)
            def _():
                pltpu.sync_copy(acc.at[pl.ds(kk * zch, zch)], ub0)
                pltpu.sync_copy(ub0, out_hbm.at[cc, pl.ds(kk * zch, zch)])

    return k(m, dst, zrows_hbm)


def _dot(a, b):
    return jax.lax.dot_general(a, b, (((1,), (0,)), ((), ())),
                               preferred_element_type=jnp.float32)


def _dot0(a, b):
    return jax.lax.dot_general(a, b, (((0,), (0,)), ((), ())),
                               preferred_element_type=jnp.float32)


def _pack(x):
    r, c = x.shape
    k = 128 // c
    x3 = x.reshape(r // k, k, c)
    return jnp.concatenate([x3[:, j] for j in range(k)], axis=1)


def _unpack(p, c):
    q = p.shape[0]
    k = 128 // c
    g3 = jnp.stack([p[:, j * c:(j + 1) * c] for j in range(k)], axis=1)
    return g3.reshape(q * k, c)


def _full(shape):
    return pl.BlockSpec(shape, lambda i: tuple(0 for _ in shape))


def _prep1(xT, tW1, tb1, tW2, tb2, w1a, b1, bn):
    n = xT.shape[1]

    def body(x_ref, tw1, tb1r, tw2, tb2r, wa, b1r, x0_ref, g1_ref):
        x0 = jnp.maximum(_dot0(x_ref[...], tw1[...]) + tb1r[...], 0.0)
        x0 = jnp.maximum(_dot(x0, tw2[...]) + tb2r[...], 0.0)
        x0_ref[...] = x0
        g1 = _dot(x0, wa[...]) + b1r[...]
        g1_ref[...] = _pack(g1)

    return pl.pallas_call(
        body,
        grid=(pl.cdiv(n, bn),),
        in_specs=[
            pl.BlockSpec((xT.shape[0], bn), lambda i: (0, i)),
            _full(tW1.shape), _full(tb1.shape), _full(tW2.shape),
            _full(tb2.shape), _full(w1a.shape), _full(b1.shape),
        ],
        out_specs=[
            pl.BlockSpec((bn, 32), lambda i: (i, 0)),
            pl.BlockSpec((bn // 4, 128), lambda i: (i, 0)),
        ],
        out_shape=[
            jax.ShapeDtypeStruct((n, 32), jnp.float32),
            jax.ShapeDtypeStruct((n // 4, 128), jnp.float32),
        ],
    )(xT, tW1, tb1, tW2, tb2, w1a, b1)


def _edge_mlp(garrp, eaT, w1e, w2, b2, be, c):
    e = eaT.shape[1]
    gr = be * c // 128

    def body(g_ref, ea_ref, we, w2r, b2r, out_ref):
        g = _unpack(g_ref[...], c)
        h = jnp.maximum(g + _dot0(ea_ref[...], we[...]), 0.0)
        m = jnp.maximum(_dot(h, w2r[...]) + b2r[...], 0.0)
        out_ref[0] = _pack(m[:, :32])
        out_ref[1] = _pack(m[:, 32:])

    return pl.pallas_call(
        body,
        grid=(pl.cdiv(e, be),),
        in_specs=[
            pl.BlockSpec((gr, 128), lambda i: (i, 0)),
            pl.BlockSpec((16, be), lambda i: (0, i)),
            _full(w1e.shape), _full(w2.shape), _full(b2.shape),
        ],
        out_specs=pl.BlockSpec((2, be // 4, 128), lambda i: (0, i, 0)),
        out_shape=jax.ShapeDtypeStruct((2, e // 4, 128), jnp.float32),
    )(garrp, eaT, w1e, w2, b2)


def _mid(aggp, deg0, deg1, x0, wlo, whi, wx, b1, bn):
    n = x0.shape[0]

    def body(a_ref, d0, d1, x0_ref, wl, wh, wxr, b1r, out_ref):
        inv = 1.0 / jnp.maximum(d0[...] + d1[...], 1.0)
        a = a_ref[...]
        lo = _unpack(a[0], 32) * inv
        hi = _unpack(a[1], 32) * inv
        g2 = (_dot(lo, wl[...]) + _dot(hi, wh[...])
              + _dot(x0_ref[...], wxr[...]) + b1r[...])
        out_ref[...] = _pack(g2)

    return pl.pallas_call(
        body,
        grid=(pl.cdiv(n, bn),),
        in_specs=[
            pl.BlockSpec((2, bn // 4, 128), lambda i: (0, i, 0)),
            pl.BlockSpec((bn, 1), lambda i: (i, 0)),
            pl.BlockSpec((bn, 1), lambda i: (i, 0)),
            pl.BlockSpec((bn, 32), lambda i: (i, 0)),
            _full(wlo.shape), _full(whi.shape), _full(wx.shape),
            _full(b1.shape),
        ],
        out_specs=pl.BlockSpec((bn // 2, 128), lambda i: (i, 0)),
        out_shape=jax.ShapeDtypeStruct((n // 2, 128), jnp.float32),
    )(aggp, deg0, deg1, x0, wlo, whi, wx, b1)


def _head(aggp, deg0, deg1, x0, plo, phi, px, pb1, pw2, pb2, bn):
    n = x0.shape[0]

    def body(a_ref, d0, d1, x0_ref, wl, wh, wxr, b1r, w2r, b2r, out_ref):
        inv = 1.0 / jnp.maximum(d0[...] + d1[...], 1.0)
        a = a_ref[...]
        lo = _unpack(a[0], 32) * inv
        hi = _unpack(a[1], 32) * inv
        h = jnp.maximum(_dot(lo, wl[...]) + _dot(hi, wh[...])
                        + _dot(x0_ref[...], wxr[...]) + b1r[...], 0.0)
        out_ref[...] = _dot(h, w2r[...]) + b2r[...]

    return pl.pallas_call(
        body,
        grid=(pl.cdiv(n, bn),),
        in_specs=[
            pl.BlockSpec((2, bn // 4, 128), lambda i: (0, i, 0)),
            pl.BlockSpec((bn, 1), lambda i: (i, 0)),
            pl.BlockSpec((bn, 1), lambda i: (i, 0)),
            pl.BlockSpec((bn, 32), lambda i: (i, 0)),
            _full(plo.shape), _full(phi.shape), _full(px.shape),
            _full(pb1.shape), _full(pw2.shape), _full(pb2.shape),
        ],
        out_specs=pl.BlockSpec((bn, 1), lambda i: (i, 0)),
        out_shape=jax.ShapeDtypeStruct((n, 1), jnp.float32),
    )(aggp, deg0, deg1, x0, plo, phi, px, pb1, pw2, pb2)


def kernel(x, sx, edge_index, edge_attr, batch, tW1, tb1, tW2, tb2,
           s1W1, s1b1, s1W2, s1b2, s2W1, s2b1, s2W2, s2b2,
           pW1, pb1, pW2, pb2):
    n = x.shape[0]
    e = edge_attr.shape[0]
    src = edge_index[0]
    dst = edge_index[1]

    bn = 2048
    be = 2048

    ones1 = jnp.ones((1000,), jnp.float32)
    z1 = jnp.zeros((2000,), jnp.float32)
    z32 = jnp.zeros((400, 32), jnp.float32)

    r = lambda v: v.reshape(1, -1)

    degp0, degp1 = _sc_degree(dst, ones1, z1, n)
    deg0 = degp0.reshape(n, 1)
    deg1 = degp1.reshape(n, 1)

    eaT = edge_attr.T

    x0, g1p = _prep1(x.T, tW1, r(tb1), tW2, r(tb2), s1W1[:32], r(s1b1), bn)

    garr1 = _sc_gather(g1p.reshape(n, 32), src, 1000)
    m1 = _edge_mlp(garr1.reshape(e // 4, 128), eaT, s1W1[32:48], s1W2,
                   r(s1b2), be, 32)
    agg1 = _sc_scatter(m1.reshape(2, e, 32), dst, z32, n)

    g2p = _mid(agg1.reshape(2, n // 4, 128), deg0, deg1, x0, s2W1[:32],
               s2W1[32:64], s2W1[64:96], r(s2b1), bn)

    garr2 = _sc_gather(g2p.reshape(n, 64), src, 1000)
    m2 = _edge_mlp(garr2.reshape(e // 2, 128), eaT, s2W1[96:112], s2W2,
                   r(s2b2), be, 64)
    agg2 = _sc_scatter(m2.reshape(2, e, 32), dst, z32, n)

    return _head(agg2.reshape(2, n // 4, 128), deg0, deg1, x0, pW1[:32],
                 pW1[32:64], pW1[64:96], r(pb1), pW2, r(pb2), bn)


# speedup vs baseline: 3.8754x; 1.3983x over previous
import functools

import jax
import jax.numpy as jnp
from jax import lax
from jax.experimental import pallas as pl
from jax.experimental.pallas import tpu as pltpu
from jax.experimental.pallas import tpu_sc as plsc

_NC = 2
_NS = 16
_NW = _NC * _NS


def _mesh():
    return plsc.VectorSubcoreMesh(core_axis_name="c", subcore_axis_name="s")


def _sc_gather(table, idx, ch):
    n, c = table.shape
    e = idx.shape[0]
    per_w = e // _NW
    assert per_w * _NW == e and per_w % ch == 0
    ng = per_w // ch

    @functools.partial(
        pl.kernel,
        out_type=jax.ShapeDtypeStruct((e, c), jnp.float32),
        mesh=_mesh(),
        compiler_params=pltpu.CompilerParams(use_tc_tiling_on_sc=False),
        scratch_types=[
            pltpu.VMEM((ch,), jnp.int32),
            pltpu.VMEM((ch,), jnp.int32),
            pltpu.VMEM((ch, c), jnp.float32),
            pltpu.VMEM((ch, c), jnp.float32),
            pltpu.SemaphoreType.DMA, pltpu.SemaphoreType.DMA,
            pltpu.SemaphoreType.DMA, pltpu.SemaphoreType.DMA,
            pltpu.SemaphoreType.DMA, pltpu.SemaphoreType.DMA,
        ],
    )
    def k(tab_hbm, idx_hbm, out_hbm, ib0, ib1, rb0, rb1,
          is0, is1, gs0, gs1, ss0, ss1):
        cc = lax.axis_index("c")
        ss = lax.axis_index("s")
        base = (ss * _NC + cc) * per_w
        ib = (ib0, ib1)
        rb = (rb0, rb1)
        isem = (is0, is1)
        gsem = (gs0, gs1)
        ssem = (ss0, ss1)

        def load_idx(g, b):
            return pltpu.async_copy(
                idx_hbm.at[pl.ds(base + g * ch, ch)], ib[b], isem[b])

        def start_gather(b):
            return pltpu.async_copy(tab_hbm.at[ib[b]], rb[b], gsem[b])

        def start_store(g, b):
            return pltpu.async_copy(
                rb[b], out_hbm.at[pl.ds(base + g * ch, ch)], ssem[b])

        pend_i = {0: load_idx(0, 0)}
        if ng > 1:
            pend_i[1] = load_idx(1, 1)
        pend_i[0].wait()
        pend_g = {0: start_gather(0)}
        pend_s = {}
        for g in range(ng):
            b = g & 1
            pend_g[g].wait()
            if g + 2 < ng:
                pend_i[g + 2] = load_idx(g + 2, b)
            if g + 1 < ng:
                pend_i[g + 1].wait()
                if g >= 1:
                    pend_s[g - 1].wait()
                pend_g[g + 1] = start_gather(1 - b)
            pend_s[g] = start_store(g, b)
        pend_s[ng - 1].wait()
        if ng > 1:
            pend_s[ng - 2].wait()

    return k(table, idx)


def _sc_degree(dst, ones_hbm, zeros_hbm, n):
    e = dst.shape[0]
    ch = 1000
    per_t = e // _NW
    assert per_t % ch == 0
    ng = per_t // ch
    zch = zeros_hbm.shape[0]
    nz = n // zch
    assert nz * zch == n

    @functools.partial(
        pl.kernel,
        out_type=[jax.ShapeDtypeStruct((n,), jnp.float32),
                  jax.ShapeDtypeStruct((n,), jnp.float32)],
        mesh=_mesh(),
        scratch_types=[
            pltpu.VMEM_SHARED((n,), jnp.float32),
            pltpu.VMEM((ch,), jnp.float32),
            pltpu.VMEM((zch,), jnp.float32),
            pltpu.VMEM((ch,), jnp.int32),
            pltpu.VMEM((ch,), jnp.int32),
            pltpu.SemaphoreType.DMA, pltpu.SemaphoreType.DMA,
        ],
    )
    def k(dst_hbm, ones_h, zero_h, out0_hbm, out1_hbm, acc, ones_v, stage,
          ib0, ib1, is0, is1):
        cc = lax.axis_index("c")
        tt = lax.axis_index("s")
        pltpu.sync_copy(ones_h, ones_v)
        pltpu.sync_copy(zero_h, stage)
        for j in range((nz + _NS - 1) // _NS):
            kk = tt + _NS * j

            @pl.when(kk < nz)
            def _():
                pltpu.sync_copy(stage, acc.at[pl.ds(kk * zch, zch)])
        plsc.subcore_barrier()

        base = cc * (e // _NC) + tt * per_t
        ib = (ib0, ib1)
        isem = (is0, is1)

        def load_idx(g, b):
            return pltpu.async_copy(
                dst_hbm.at[pl.ds(base + g * ch, ch)], ib[b], isem[b])

        pend = {0: load_idx(0, 0)}
        if ng > 1:
            pend[1] = load_idx(1, 1)
        for g in range(ng):
            b = g & 1
            pend[g].wait()
            pltpu.sync_copy(ones_v, acc.at[ib[b]], add=True)
            if g + 2 < ng:
                pend[g + 2] = load_idx(g + 2, b)
        plsc.subcore_barrier()
        for j in range((nz + _NS - 1) // _NS):
            kk = tt + _NS * j

            @pl.when(jnp.logical_and(kk < nz, cc == 0))
            def _():
                pltpu.sync_copy(acc.at[pl.ds(kk * zch, zch)], stage)
                pltpu.sync_copy(stage, out0_hbm.at[pl.ds(kk * zch, zch)])

            @pl.when(jnp.logical_and(kk < nz, cc == 1))
            def _():
                pltpu.sync_copy(acc.at[pl.ds(kk * zch, zch)], stage)
                pltpu.sync_copy(stage, out1_hbm.at[pl.ds(kk * zch, zch)])

    return k(dst, ones_hbm, zeros_hbm)


def _sc_scatter(m, dst, zrows_hbm, n):
    e = dst.shape[0]
    hc = m.shape[2]
    ch = 400
    per_t = e // _NS
    assert per_t % ch == 0
    ng = per_t // ch
    zch = zrows_hbm.shape[0]
    nz = n // zch
    assert nz * zch == n

    @functools.partial(
        pl.kernel,
        out_type=jax.ShapeDtypeStruct((_NC, n, hc), jnp.float32),
        mesh=_mesh(),
        compiler_params=pltpu.CompilerParams(use_tc_tiling_on_sc=False),
        scratch_types=[
            pltpu.VMEM_SHARED((n, hc), jnp.float32),
            pltpu.VMEM((ch,), jnp.int32),
            pltpu.VMEM((ch,), jnp.int32),
            pltpu.VMEM((ch, hc), jnp.float32),
            pltpu.VMEM((ch, hc), jnp.float32),
            pltpu.SemaphoreType.DMA, pltpu.SemaphoreType.DMA,
            pltpu.SemaphoreType.DMA, pltpu.SemaphoreType.DMA,
        ],
    )
    def k(m_hbm, dst_hbm, zrows_h, out_hbm, acc, ib0, ib1, ub0, ub1,
          is0, is1, us0, us1):
        cc = lax.axis_index("c")
        tt = lax.axis_index("s")
        pltpu.sync_copy(zrows_h, ub0)
        for j in range((nz + _NS - 1) // _NS):
            kk = tt + _NS * j

            @pl.when(kk < nz)
            def _():
                pltpu.sync_copy(ub0, acc.at[pl.ds(kk * zch, zch)])
        plsc.subcore_barrier()

        base = tt * per_t
        ib = (ib0, ib1)
        ub = (ub0, ub1)
        isem = (is0, is1)
        usem = (us0, us1)

        def load(g, b):
            di = pltpu.async_copy(
                dst_hbm.at[pl.ds(base + g * ch, ch)], ib[b], isem[b])
            du = pltpu.async_copy(
                m_hbm.at[cc, pl.ds(base + g * ch, ch)], ub[b], usem[b])
            return di, du

        pend = {0: load(0, 0)}
        if ng > 1:
            pend[1] = load(1, 1)
        for g in range(ng):
            b = g & 1
            pend[g][0].wait()
            pend[g][1].wait()
            pltpu.sync_copy(ub[b], acc.at[ib[b]], add=True)
            if g + 2 < ng:
                pend[g + 2] = load(g + 2, b)
        plsc.subcore_barrier()
        for j in range((nz + _NS - 1) // _NS):
            kk = tt + _NS * j

            @pl.when(kk < nz)
            def _():
                pltpu.sync_copy(acc.at[pl.ds(kk * zch, zch)], ub0)
                pltpu.sync_copy(ub0, out_hbm.at[cc, pl.ds(kk * zch, zch)])

    return k(m, dst, zrows_hbm)


def _dot(a, b):
    return jax.lax.dot_general(a, b, (((1,), (0,)), ((), ())),
                               preferred_element_type=jnp.float32)


def _dot0(a, b):
    return jax.lax.dot_general(a, b, (((0,), (0,)), ((), ())),
                               preferred_element_type=jnp.float32)


def _pack(x):
    r, c = x.shape
    k = 128 // c
    x3 = x.reshape(r // k, k, c)
    return jnp.concatenate([x3[:, j] for j in range(k)], axis=1)


def _unpack(p, c):
    q = p.shape[0]
    k = 128 // c
    g3 = jnp.stack([p[:, j * c:(j + 1) * c] for j in range(k)], axis=1)
    return g3.reshape(q * k, c)


def _full(shape):
    return pl.BlockSpec(shape, lambda i: tuple(0 for _ in shape))


def _prep1(xT, tW1, tb1, tW2, tb2, w1a, b1, bn):
    n = xT.shape[1]

    def body(x_ref, tw1, tb1r, tw2, tb2r, wa, b1r, x0_ref, g1_ref):
        x0 = jnp.maximum(_dot0(x_ref[...], tw1[...]) + tb1r[...], 0.0)
        x0 = jnp.maximum(_dot(x0, tw2[...]) + tb2r[...], 0.0)
        x0_ref[...] = x0
        g1 = _dot(x0, wa[...]) + b1r[...]
        g1_ref[...] = _pack(g1)

    return pl.pallas_call(
        body,
        grid=(pl.cdiv(n, bn),),
        in_specs=[
            pl.BlockSpec((xT.shape[0], bn), lambda i: (0, i)),
            _full(tW1.shape), _full(tb1.shape), _full(tW2.shape),
            _full(tb2.shape), _full(w1a.shape), _full(b1.shape),
        ],
        out_specs=[
            pl.BlockSpec((bn, 32), lambda i: (i, 0)),
            pl.BlockSpec((bn // 4, 128), lambda i: (i, 0)),
        ],
        out_shape=[
            jax.ShapeDtypeStruct((n, 32), jnp.float32),
            jax.ShapeDtypeStruct((n // 4, 128), jnp.float32),
        ],
    )(xT, tW1, tb1, tW2, tb2, w1a, b1)


def _edge_mlp(garrp, eaR, w1big, w2lo, w2hi, b2lo, b2hi, aub):
    e4, gw = garrp.shape

    def body(g_ref, ea_ref, w1, wlo, whi, blo, bhi, out_ref):
        et = _dot0(ea_ref[...], w1[...])
        h = jnp.maximum(g_ref[...] + et, 0.0)
        out_ref[0] = jnp.maximum(_dot(h, wlo[...]) + blo[...], 0.0)
        out_ref[1] = jnp.maximum(_dot(h, whi[...]) + bhi[...], 0.0)

    return pl.pallas_call(
        body,
        grid=(pl.cdiv(e4, aub),),
        in_specs=[
            pl.BlockSpec((aub, gw), lambda i: (i, 0)),
            pl.BlockSpec((64, aub), lambda i: (0, i)),
            _full(w1big.shape), _full(w2lo.shape), _full(w2hi.shape),
            _full(b2lo.shape), _full(b2hi.shape),
        ],
        out_specs=pl.BlockSpec((2, aub, 128), lambda i: (0, i, 0)),
        out_shape=jax.ShapeDtypeStruct((2, e4, 128), jnp.float32),
    )(garrp, eaR, w1big, w2lo, w2hi, b2lo, b2hi)


def _mid(aggp, deg0, deg1, x0, wlo, whi, wx, b1, bn):
    n = x0.shape[0]

    def body(a_ref, d0, d1, x0_ref, wl, wh, wxr, b1r, out_ref):
        inv = 1.0 / jnp.maximum(d0[...] + d1[...], 1.0)
        a = a_ref[...]
        lo = _unpack(a[0], 32) * inv
        hi = _unpack(a[1], 32) * inv
        g2 = (_dot(lo, wl[...]) + _dot(hi, wh[...])
              + _dot(x0_ref[...], wxr[...]) + b1r[...])
        out_ref[...] = _pack(g2)

    return pl.pallas_call(
        body,
        grid=(pl.cdiv(n, bn),),
        in_specs=[
            pl.BlockSpec((2, bn // 4, 128), lambda i: (0, i, 0)),
            pl.BlockSpec((bn, 1), lambda i: (i, 0)),
            pl.BlockSpec((bn, 1), lambda i: (i, 0)),
            pl.BlockSpec((bn, 32), lambda i: (i, 0)),
            _full(wlo.shape), _full(whi.shape), _full(wx.shape),
            _full(b1.shape),
        ],
        out_specs=pl.BlockSpec((bn // 2, 128), lambda i: (i, 0)),
        out_shape=jax.ShapeDtypeStruct((n // 2, 128), jnp.float32),
    )(aggp, deg0, deg1, x0, wlo, whi, wx, b1)


def _head(aggp, deg0, deg1, x0, plo, phi, px, pb1, pw2, pb2, bn):
    n = x0.shape[0]

    def body(a_ref, d0, d1, x0_ref, wl, wh, wxr, b1r, w2r, b2r, out_ref):
        inv = 1.0 / jnp.maximum(d0[...] + d1[...], 1.0)
        a = a_ref[...]
        lo = _unpack(a[0], 32) * inv
        hi = _unpack(a[1], 32) * inv
        h = jnp.maximum(_dot(lo, wl[...]) + _dot(hi, wh[...])
                        + _dot(x0_ref[...], wxr[...]) + b1r[...], 0.0)
        out_ref[...] = _dot(h, w2r[...]) + b2r[...]

    return pl.pallas_call(
        body,
        grid=(pl.cdiv(n, bn),),
        in_specs=[
            pl.BlockSpec((2, bn // 4, 128), lambda i: (0, i, 0)),
            pl.BlockSpec((bn, 1), lambda i: (i, 0)),
            pl.BlockSpec((bn, 1), lambda i: (i, 0)),
            pl.BlockSpec((bn, 32), lambda i: (i, 0)),
            _full(plo.shape), _full(phi.shape), _full(px.shape),
            _full(pb1.shape), _full(pw2.shape), _full(pb2.shape),
        ],
        out_specs=pl.BlockSpec((bn, 1), lambda i: (i, 0)),
        out_shape=jax.ShapeDtypeStruct((n, 1), jnp.float32),
    )(aggp, deg0, deg1, x0, plo, phi, px, pb1, pw2, pb2)


def kernel(x, sx, edge_index, edge_attr, batch, tW1, tb1, tW2, tb2,
           s1W1, s1b1, s1W2, s1b2, s2W1, s2b1, s2W2, s2b2,
           pW1, pb1, pW2, pb2):
    n = x.shape[0]
    e = edge_attr.shape[0]
    src = edge_index[0]
    dst = edge_index[1]

    bn = 2048
    be = 2048

    ones1 = jnp.ones((1000,), jnp.float32)
    z1 = jnp.zeros((2000,), jnp.float32)
    z32 = jnp.zeros((400, 32), jnp.float32)

    r = lambda v: v.reshape(1, -1)

    degp0, degp1 = _sc_degree(dst, ones1, z1, n)
    deg0 = degp0.reshape(n, 1)
    deg1 = degp1.reshape(n, 1)

    e4 = e // 4
    eaR = edge_attr.T.reshape(64, e4)
    srcp = src.reshape(4, e4).T.reshape(e)
    dstp = dst.reshape(4, e4).T.reshape(e)
    i4 = jnp.eye(4, dtype=jnp.float32)

    def bd1(w):
        return (w[:, None, None, :] * i4[None, :, :, None]).reshape(
            64, 4 * w.shape[1])

    def bd2(w):
        return (i4[:, None, :, None] * w[None, :, None, :]).reshape(
            4 * w.shape[0], 128)

    def t4(b):
        return jnp.tile(b, 4).reshape(1, 128)

    x0, g1p = _prep1(x.T, tW1, r(tb1), tW2, r(tb2), s1W1[:32], r(s1b1), bn)

    garr1 = _sc_gather(g1p.reshape(n, 32), srcp, 1000)
    m1 = _edge_mlp(garr1.reshape(e4, 128), eaR, bd1(s1W1[32:48]),
                   bd2(s1W2[:, :32]), bd2(s1W2[:, 32:]),
                   t4(s1b2[:32]), t4(s1b2[32:]), 1024)
    agg1 = _sc_scatter(m1.reshape(2, e, 32), dstp, z32, n)

    g2p = _mid(agg1.reshape(2, n // 4, 128), deg0, deg1, x0, s2W1[:32],
               s2W1[32:64], s2W1[64:96], r(s2b1), bn)

    garr2 = _sc_gather(g2p.reshape(n, 64), srcp, 1000)
    m2 = _edge_mlp(garr2.reshape(e4, 256), eaR, bd1(s2W1[96:112]),
                   bd2(s2W2[:, :32]), bd2(s2W2[:, 32:]),
                   t4(s2b2[:32]), t4(s2b2[32:]), 1024)
    agg2 = _sc_scatter(m2.reshape(2, e, 32), dstp, z32, n)

    return _head(agg2.reshape(2, n // 4, 128), deg0, deg1, x0, pW1[:32],
                 pW1[32:64], pW1[64:96], r(pb1), pW2, r(pb2), bn)
